# Initial kernel scaffold; baseline (speedup 1.0000x reference)
import jax
import jax.numpy as jnp
from jax.experimental import pallas as pl


def kernel(x, genre_emb, gate_w, genre_proj_w, genre_proj_b):
    raise NotImplementedError("write your pallas kernel here")


# speedup vs baseline: 1.0158x; 1.0158x over previous
import functools

import jax
import jax.numpy as jnp
from jax import lax
from jax.experimental import pallas as pl
from jax.experimental.pallas import tpu as pltpu


def _gate_kernel(x_ref, ge_ref, wt_ref, gwt_ref, b_ref, w_ref, i_ref,
                 *, L, E, Tb):
    ge_bf = ge_ref[...].astype(jnp.bfloat16)
    p = lax.dot_general(ge_bf, wt_ref[...], (((1,), (0,)), ((), ())),
                        preferred_element_type=jnp.float32)
    h = x_ref[...] + p + b_ref[...]
    lg_t = lax.dot_general(h.astype(jnp.bfloat16), gwt_ref[...],
                           (((1,), (0,)), ((), ())),
                           preferred_element_type=jnp.float32)
    lg = jnp.transpose(lg_t, (1, 0)).reshape(L, E, Tb)

    eiota = lax.broadcasted_iota(jnp.int32, (L, E, Tb), 1)
    top1 = jnp.max(lg, axis=1)
    i1 = jnp.min(jnp.where(lg == top1[:, None, :], eiota, E), axis=1)
    masked = jnp.where(eiota == i1[:, None, :], -jnp.inf, lg)
    top2 = jnp.max(masked, axis=1)
    i2 = jnp.min(jnp.where(masked == top2[:, None, :], eiota, E), axis=1)

    w1 = jax.nn.sigmoid(top1 - top2)
    w2 = jax.nn.sigmoid(top2 - top1)
    w_ref[...] = jnp.stack([w1, w2])
    i_ref[...] = jnp.stack([i1, i2])


def kernel(x, genre_emb, gate_w, genre_proj_w, genre_proj_b):
    T, D = x.shape
    Lyr, E, _ = gate_w.shape
    LE = Lyr * E

    wt = genre_proj_w.T.astype(jnp.bfloat16)
    gwt = gate_w.reshape(LE, D).T.astype(jnp.bfloat16)
    b2 = genre_proj_b.reshape(1, D)

    Tb = 256
    w_out, i_out = pl.pallas_call(
        functools.partial(_gate_kernel, L=Lyr, E=E, Tb=Tb),
        grid=(T // Tb,),
        in_specs=[
            pl.BlockSpec((Tb, D), lambda i: (i, 0)),
            pl.BlockSpec((Tb, D), lambda i: (i, 0)),
            pl.BlockSpec((D, D), lambda i: (0, 0)),
            pl.BlockSpec((D, LE), lambda i: (0, 0)),
            pl.BlockSpec((1, D), lambda i: (0, 0)),
        ],
        out_specs=[
            pl.BlockSpec((2, Lyr, Tb), lambda i: (0, 0, i)),
            pl.BlockSpec((2, Lyr, Tb), lambda i: (0, 0, i)),
        ],
        out_shape=[
            jax.ShapeDtypeStruct((2, Lyr, T), jnp.float32),
            jax.ShapeDtypeStruct((2, Lyr, T), jnp.int32),
        ],
    )(x, genre_emb, wt, gwt, b2)

    routing_weights = jnp.transpose(w_out, (1, 2, 0))
    expert_indices = jnp.transpose(i_out, (1, 2, 0))
    return routing_weights, expert_indices


# speedup vs baseline: 1.1062x; 1.0891x over previous
import functools

import jax
import jax.numpy as jnp
from jax import lax
from jax.experimental import pallas as pl
from jax.experimental.pallas import tpu as pltpu


def _cast_kernel(w_ref, o_ref):
    o_ref[...] = w_ref[...].astype(jnp.bfloat16)


def _gate_kernel(x_ref, ge_ref, wbf_ref, gw_ref, b_ref, w_out_ref, i_out_ref,
                 *, L, E, Tb):
    ge_bf = ge_ref[...].astype(jnp.bfloat16)
    p = lax.dot_general(ge_bf, wbf_ref[...], (((1,), (1,)), ((), ())),
                        preferred_element_type=jnp.float32)
    h = x_ref[...] + p + b_ref[...]
    lg_t = lax.dot_general(h.astype(jnp.bfloat16),
                           gw_ref[...].astype(jnp.bfloat16),
                           (((1,), (1,)), ((), ())),
                           preferred_element_type=jnp.float32)
    lg = jnp.transpose(lg_t, (1, 0)).reshape(L, E, Tb)

    eiota = lax.broadcasted_iota(jnp.int32, (L, E, Tb), 1)
    top1 = jnp.max(lg, axis=1)
    i1 = jnp.min(jnp.where(lg == top1[:, None, :], eiota, E), axis=1)
    masked = jnp.where(eiota == i1[:, None, :], -jnp.inf, lg)
    top2 = jnp.max(masked, axis=1)
    i2 = jnp.min(jnp.where(masked == top2[:, None, :], eiota, E), axis=1)

    w1 = jax.nn.sigmoid(top1 - top2)
    w2 = jax.nn.sigmoid(top2 - top1)
    w_out_ref[...] = jnp.stack([w1, w2])
    i_out_ref[...] = jnp.stack([i1, i2])


def kernel(x, genre_emb, gate_w, genre_proj_w, genre_proj_b):
    T, D = x.shape
    Lyr, E, _ = gate_w.shape
    LE = Lyr * E

    CB = 512
    wbf = pl.pallas_call(
        _cast_kernel,
        grid=(D // CB,),
        in_specs=[pl.BlockSpec((CB, D), lambda i: (i, 0))],
        out_specs=pl.BlockSpec((CB, D), lambda i: (i, 0)),
        out_shape=jax.ShapeDtypeStruct((D, D), jnp.bfloat16),
    )(genre_proj_w)

    gw2 = gate_w.reshape(LE, D)
    b2 = genre_proj_b.reshape(1, D)

    Tb = 256
    w_out, i_out = pl.pallas_call(
        functools.partial(_gate_kernel, L=Lyr, E=E, Tb=Tb),
        grid=(T // Tb,),
        in_specs=[
            pl.BlockSpec((Tb, D), lambda i: (i, 0)),
            pl.BlockSpec((Tb, D), lambda i: (i, 0)),
            pl.BlockSpec((D, D), lambda i: (0, 0)),
            pl.BlockSpec((LE, D), lambda i: (0, 0)),
            pl.BlockSpec((1, D), lambda i: (0, 0)),
        ],
        out_specs=[
            pl.BlockSpec((2, Lyr, Tb), lambda i: (0, 0, i)),
            pl.BlockSpec((2, Lyr, Tb), lambda i: (0, 0, i)),
        ],
        out_shape=[
            jax.ShapeDtypeStruct((2, Lyr, T), jnp.float32),
            jax.ShapeDtypeStruct((2, Lyr, T), jnp.int32),
        ],
    )(x, genre_emb, wbf, gw2, b2)

    routing_weights = jnp.transpose(w_out, (1, 2, 0))
    expert_indices = jnp.transpose(i_out, (1, 2, 0))
    return routing_weights, expert_indices
